# baseline (speedup 1.0000x reference)
import jax
import jax.numpy as jnp
from jax import lax
from jax.experimental import pallas as pl
from jax.experimental.pallas import tpu as pltpu
from jax.experimental.pallas import tpu_sc as plsc

_B = 16384
_GL = 6
_NC = 2
_NS = 16
_NW = _NC * _NS
_BPW = _B // _NW
_C = 128
_NCH = _BPW // _C


def _body(uid_h, mov_h, gen_h, age_h, occ_h, zip_h, gent_h,
          w_uid, w_mov, w_gen, w_age, w_occ, w_zip, w_gnr,
          o_uid, o_mov, o_gen, o_age, o_occ, o_zip, o_gnr,
          i_uid, i_mov, i_gen, i_age, i_occ, i_zip, i_gnr,
          r_uid, r_mov, r_gen, r_age, r_occ, r_zip, r_gnr, pooled,
          sem):
  cid = lax.axis_index("c")
  sid = lax.axis_index("s")
  wid = sid * _NC + cid

  def chunk(k, carry):
    base = wid * _BPW + k * _C

    pltpu.sync_copy(uid_h.at[pl.ds(base, _C)], i_uid)
    pltpu.sync_copy(mov_h.at[pl.ds(base, _C)], i_mov)
    pltpu.sync_copy(gen_h.at[pl.ds(base, _C)], i_gen)
    pltpu.sync_copy(age_h.at[pl.ds(base, _C)], i_age)
    pltpu.sync_copy(occ_h.at[pl.ds(base, _C)], i_occ)
    pltpu.sync_copy(zip_h.at[pl.ds(base, _C)], i_zip)
    for g in range(_GL):
      pltpu.sync_copy(gent_h.at[g, pl.ds(base, _C)], i_gnr.at[g])

    cps = [
        pltpu.async_copy(w_uid.at[i_uid], r_uid, sem),
        pltpu.async_copy(w_mov.at[i_mov], r_mov, sem),
        pltpu.async_copy(w_gen.at[i_gen], r_gen, sem),
        pltpu.async_copy(w_age.at[i_age], r_age, sem),
        pltpu.async_copy(w_occ.at[i_occ], r_occ, sem),
        pltpu.async_copy(w_zip.at[i_zip], r_zip, sem),
    ]
    for g in range(_GL):
      cps.append(pltpu.async_copy(w_gnr.at[i_gnr.at[g]], r_gnr.at[g], sem))
    for cp in cps:
      cp.wait()

    def pool(s, c2):
      for h in range(2):
        acc = r_gnr[0, s, pl.ds(16 * h, 16)]
        for g in range(1, _GL):
          acc = acc + r_gnr[g, s, pl.ds(16 * h, 16)]
        pooled[s, pl.ds(16 * h, 16)] = acc * (1.0 / _GL)
      return c2

    lax.fori_loop(0, _C, pool, 0)

    pltpu.sync_copy(r_uid, o_uid.at[pl.ds(base, _C)])
    pltpu.sync_copy(r_mov, o_mov.at[pl.ds(base, _C)])
    pltpu.sync_copy(r_gen, o_gen.at[pl.ds(base, _C)])
    pltpu.sync_copy(r_age, o_age.at[pl.ds(base, _C)])
    pltpu.sync_copy(r_occ, o_occ.at[pl.ds(base, _C)])
    pltpu.sync_copy(r_zip, o_zip.at[pl.ds(base, _C)])
    pltpu.sync_copy(pooled, o_gnr.at[pl.ds(base, _C)])
    return carry

  lax.fori_loop(0, _NCH, chunk, 0)


@jax.jit
def _run(uid, movieid, gender, age, occ, zip_code, genres_t,
         W_uid, W_movieid, W_gender, W_age, W_occ, W_zip_code, W_genres):
  f32 = jnp.float32
  out_type = (
      jax.ShapeDtypeStruct((_B, 64), f32),
      jax.ShapeDtypeStruct((_B, 64), f32),
      jax.ShapeDtypeStruct((_B, 16), f32),
      jax.ShapeDtypeStruct((_B, 16), f32),
      jax.ShapeDtypeStruct((_B, 16), f32),
      jax.ShapeDtypeStruct((_B, 32), f32),
      jax.ShapeDtypeStruct((_B, 32), f32),
  )
  scratch_types = [
      pltpu.VMEM((_C,), jnp.int32),
      pltpu.VMEM((_C,), jnp.int32),
      pltpu.VMEM((_C,), jnp.int32),
      pltpu.VMEM((_C,), jnp.int32),
      pltpu.VMEM((_C,), jnp.int32),
      pltpu.VMEM((_C,), jnp.int32),
      pltpu.VMEM((_GL, _C), jnp.int32),
      pltpu.VMEM((_C, 64), f32),
      pltpu.VMEM((_C, 64), f32),
      pltpu.VMEM((_C, 16), f32),
      pltpu.VMEM((_C, 16), f32),
      pltpu.VMEM((_C, 16), f32),
      pltpu.VMEM((_C, 32), f32),
      pltpu.VMEM((_GL, _C, 32), f32),
      pltpu.VMEM((_C, 32), f32),
      pltpu.SemaphoreType.DMA,
  ]
  run = pl.kernel(
      _body,
      out_type=out_type,
      mesh=plsc.VectorSubcoreMesh(core_axis_name="c", subcore_axis_name="s"),
      scratch_types=scratch_types,
      compiler_params=pltpu.CompilerParams(use_tc_tiling_on_sc=False),
  )
  return run(uid, movieid, gender, age, occ, zip_code, genres_t,
             W_uid, W_movieid, W_gender, W_age, W_occ, W_zip_code, W_genres)


def kernel(uid, movieid, gender, age, occ, zip_code, genres,
           W_uid, W_movieid, W_gender, W_age, W_occ, W_zip_code, W_genres):
  i32 = jnp.int32
  genres_t = genres.astype(i32).T
  return _run(uid.astype(i32), movieid.astype(i32), gender.astype(i32),
              age.astype(i32), occ.astype(i32), zip_code.astype(i32), genres_t,
              W_uid, W_movieid, W_gender, W_age, W_occ, W_zip_code, W_genres)


# speedup vs baseline: 1.6362x; 1.6362x over previous
import jax
import jax.numpy as jnp
from jax import lax
from jax.experimental import pallas as pl
from jax.experimental.pallas import tpu as pltpu
from jax.experimental.pallas import tpu_sc as plsc

_B = 16384
_GL = 6
_NC = 2
_NS = 16
_NW = _NC * _NS
_BPW = _B // _NW
_C = 128
_NCH = _BPW // _C

_UN = 1000000
_UB = _UN // 8
_CU = 32
_NCHU = _BPW // _CU


def _mesh():
  return plsc.VectorSubcoreMesh(core_axis_name="c", subcore_axis_name="s")


def _wid():
  return lax.axis_index("s") * _NC + lax.axis_index("c")


def _uid_body(uid_h, w3d, out_h, i_vmem, gbuf, stage, sem, semo):
  wid = _wid()
  wbase = wid * _BPW
  pltpu.sync_copy(uid_h.at[pl.ds(wbase, _BPW)], i_vmem.at[pl.ds(0, _BPW)])

  def scalar_idx(s):
    return i_vmem[pl.ds(s, 16)][0]

  def chunk(k, c2):
    base = k * _CU

    def fire(s, c3):
      blk = lax.shift_right_logical(scalar_idx(base + s), 3)
      pltpu.make_async_copy(w3d.at[blk], gbuf.at[s], sem).start()
      return c3
    lax.fori_loop(0, _CU, fire, 0)

    def drain(s, c3):
      pltpu.make_async_copy(w3d.at[0], gbuf.at[s], sem).wait()
      return c3
    lax.fori_loop(0, _CU, drain, 0)

    def extract(s, c3):
      r = scalar_idx(base + s) & 7
      for h in range(4):
        stage[s, pl.ds(16 * h, 16)] = gbuf[s, r, pl.ds(16 * h, 16)]
      return c3
    lax.fori_loop(0, _CU, extract, 0)

    pltpu.sync_copy(stage, out_h.at[pl.ds(wbase + base, _CU)])
    return c2
  lax.fori_loop(0, _NCHU, chunk, 0)


def _rest_body(mov_h, gen_h, age_h, occ_h, zip_h, gent_h,
               w_mov, w_gen, w_age, w_occ, w_zip, w_gnr,
               o_mov, o_gen, o_age, o_occ, o_zip, o_gnr,
               i_mov, i_gen, i_age, i_occ, i_zip, i_gnr,
               r_mov, r_gen, r_age, r_occ, r_zip, r_gnr, pooled,
               sem):
  wid = _wid()

  def chunk(k, carry):
    base = wid * _BPW + k * _C

    pltpu.sync_copy(mov_h.at[pl.ds(base, _C)], i_mov)
    pltpu.sync_copy(gen_h.at[pl.ds(base, _C)], i_gen)
    pltpu.sync_copy(age_h.at[pl.ds(base, _C)], i_age)
    pltpu.sync_copy(occ_h.at[pl.ds(base, _C)], i_occ)
    pltpu.sync_copy(zip_h.at[pl.ds(base, _C)], i_zip)
    for g in range(_GL):
      pltpu.sync_copy(gent_h.at[g, pl.ds(base, _C)], i_gnr.at[g])

    cps = [
        pltpu.async_copy(w_mov.at[i_mov], r_mov, sem),
        pltpu.async_copy(w_gen.at[i_gen], r_gen, sem),
        pltpu.async_copy(w_age.at[i_age], r_age, sem),
        pltpu.async_copy(w_occ.at[i_occ], r_occ, sem),
        pltpu.async_copy(w_zip.at[i_zip], r_zip, sem),
    ]
    for g in range(_GL):
      cps.append(pltpu.async_copy(w_gnr.at[i_gnr.at[g]], r_gnr.at[g], sem))
    for cp in cps:
      cp.wait()

    def pool(s, c2):
      for h in range(2):
        acc = r_gnr[0, s, pl.ds(16 * h, 16)]
        for g in range(1, _GL):
          acc = acc + r_gnr[g, s, pl.ds(16 * h, 16)]
        pooled[s, pl.ds(16 * h, 16)] = acc * (1.0 / _GL)
      return c2
    lax.fori_loop(0, _C, pool, 0)

    pltpu.sync_copy(r_mov, o_mov.at[pl.ds(base, _C)])
    pltpu.sync_copy(r_gen, o_gen.at[pl.ds(base, _C)])
    pltpu.sync_copy(r_age, o_age.at[pl.ds(base, _C)])
    pltpu.sync_copy(r_occ, o_occ.at[pl.ds(base, _C)])
    pltpu.sync_copy(r_zip, o_zip.at[pl.ds(base, _C)])
    pltpu.sync_copy(pooled, o_gnr.at[pl.ds(base, _C)])
    return carry

  lax.fori_loop(0, _NCH, chunk, 0)


@jax.jit
def _run(uid, movieid, gender, age, occ, zip_code, genres_t,
         W_uid3d, W_movieid, W_gender, W_age, W_occ, W_zip_code, W_genres):
  f32 = jnp.float32

  uid_kernel = pl.kernel(
      _uid_body,
      out_type=jax.ShapeDtypeStruct((_B, 64), f32),
      mesh=_mesh(),
      scratch_types=[
          pltpu.VMEM((_BPW + 16,), jnp.int32),
          pltpu.VMEM((_CU, 8, 64), f32),
          pltpu.VMEM((_CU, 64), f32),
          pltpu.SemaphoreType.DMA,
          pltpu.SemaphoreType.DMA,
      ],
      compiler_params=pltpu.CompilerParams(use_tc_tiling_on_sc=True,
                                           needs_layout_passes=False),
  )
  out_uid = uid_kernel(uid, W_uid3d)

  rest_kernel = pl.kernel(
      _rest_body,
      out_type=(
          jax.ShapeDtypeStruct((_B, 64), f32),
          jax.ShapeDtypeStruct((_B, 16), f32),
          jax.ShapeDtypeStruct((_B, 16), f32),
          jax.ShapeDtypeStruct((_B, 16), f32),
          jax.ShapeDtypeStruct((_B, 32), f32),
          jax.ShapeDtypeStruct((_B, 32), f32),
      ),
      mesh=_mesh(),
      scratch_types=[
          pltpu.VMEM((_C,), jnp.int32),
          pltpu.VMEM((_C,), jnp.int32),
          pltpu.VMEM((_C,), jnp.int32),
          pltpu.VMEM((_C,), jnp.int32),
          pltpu.VMEM((_C,), jnp.int32),
          pltpu.VMEM((_GL, _C), jnp.int32),
          pltpu.VMEM((_C, 64), f32),
          pltpu.VMEM((_C, 16), f32),
          pltpu.VMEM((_C, 16), f32),
          pltpu.VMEM((_C, 16), f32),
          pltpu.VMEM((_C, 32), f32),
          pltpu.VMEM((_GL, _C, 32), f32),
          pltpu.VMEM((_C, 32), f32),
          pltpu.SemaphoreType.DMA,
      ],
      compiler_params=pltpu.CompilerParams(use_tc_tiling_on_sc=False),
  )
  out_mov, out_gen, out_age, out_occ, out_zip, out_gnr = rest_kernel(
      movieid, gender, age, occ, zip_code, genres_t,
      W_movieid, W_gender, W_age, W_occ, W_zip_code, W_genres)

  return (out_uid, out_mov, out_gen, out_age, out_occ, out_zip, out_gnr)


def kernel(uid, movieid, gender, age, occ, zip_code, genres,
           W_uid, W_movieid, W_gender, W_age, W_occ, W_zip_code, W_genres):
  i32 = jnp.int32
  genres_t = genres.astype(i32).T
  W_uid3d = W_uid.reshape(_UB, 8, 64)
  return _run(uid.astype(i32), movieid.astype(i32), gender.astype(i32),
              age.astype(i32), occ.astype(i32), zip_code.astype(i32), genres_t,
              W_uid3d, W_movieid, W_gender, W_age, W_occ, W_zip_code, W_genres)
